# baseline (speedup 1.0000x reference)
import jax
import jax.numpy as jnp
import numpy as np
from jax import lax
from jax.experimental import pallas as pl
from jax.experimental.pallas import tpu as pltpu
from jax.experimental.pallas import tpu_sc as plsc

_N, _E, _D, _C, _L = 10000, 320000, 128, 16, 3
_NC, _NS = 2, 16
_NW = _NC * _NS
_CHUNK = 96
_NCH = 106
_EPAD = _CHUNK * _NCH * _NW
_NPAD = 10112
_RPT = _NPAD // _NS
_BR = 1000

_PIDX = np.arange(_EPAD - _E, dtype=np.int32)
_PADS = np.stack([_PIDX % _N,
                  _N + _PIDX % (_NPAD - _N)]).astype(np.int32)
_ZEROS = np.zeros((_RPT, _D), np.float32)


def _sc_scatter_body(m_hbm, src_hbm, dst_hbm, zeros_hbm, out0, out1,
                     src_v, dst_v, rows0, rows1, sem0a, sem0b, sem1a, sem1b,
                     agg_sp):
    c = lax.axis_index("c")
    s = lax.axis_index("s")
    wid = s * _NC + c
    row0 = s * _RPT
    pltpu.sync_copy(zeros_hbm, agg_sp.at[pl.ds(row0, _RPT)])
    pltpu.sync_copy(src_hbm.at[wid], src_v)
    pltpu.sync_copy(dst_hbm.at[wid], dst_v)
    plsc.subcore_barrier()

    _H = _CHUNK // 2

    def _issue(j, rv, sa, sb):
        pltpu.async_copy(
            m_hbm.at[src_v.at[pl.ds(j * _CHUNK, _H)]], rv.at[pl.ds(0, _H)], sa)
        pltpu.async_copy(
            m_hbm.at[src_v.at[pl.ds(j * _CHUNK + _H, _H)]],
            rv.at[pl.ds(_H, _H)], sb)

    def _wait(j, rv, sa, sb):
        pltpu.make_async_copy(
            m_hbm.at[src_v.at[pl.ds(j * _CHUNK, _H)]], rv.at[pl.ds(0, _H)],
            sa).wait()
        pltpu.make_async_copy(
            m_hbm.at[src_v.at[pl.ds(j * _CHUNK + _H, _H)]],
            rv.at[pl.ds(_H, _H)], sb).wait()

    _issue(0, rows0, sem0a, sem0b)

    def body(g, carry):
        for b, rv, sa, sb, rvn, sna, snb in (
                (0, rows0, sem0a, sem0b, rows1, sem1a, sem1b),
                (1, rows1, sem1a, sem1b, rows0, sem0a, sem0b)):
            j = 2 * g + b
            nxt = j + 1

            @pl.when(nxt < _NCH)
            def _(rvn=rvn, sna=sna, snb=snb, nxt=nxt):
                _issue(nxt, rvn, sna, snb)

            _wait(j, rv, sa, sb)
            pltpu.sync_copy(rv, agg_sp.at[dst_v.at[j]], add=True)
        return carry

    lax.fori_loop(0, _NCH // 2, body, 0)
    plsc.subcore_barrier()

    @pl.when(c == 0)
    def _():
        pltpu.sync_copy(agg_sp.at[pl.ds(row0, _RPT)], out0.at[pl.ds(row0, _RPT)])

    @pl.when(c == 1)
    def _():
        pltpu.sync_copy(agg_sp.at[pl.ds(row0, _RPT)], out1.at[pl.ds(row0, _RPT)])


def _sc_scatter(m, src_t, dst_t, zeros):
    f = pl.kernel(
        _sc_scatter_body,
        out_type=(jax.ShapeDtypeStruct((_NPAD, _D), jnp.float32),
                  jax.ShapeDtypeStruct((_NPAD, _D), jnp.float32)),
        mesh=plsc.VectorSubcoreMesh(core_axis_name="c", subcore_axis_name="s"),
        scratch_types=[
            pltpu.VMEM((_NCH * _CHUNK,), jnp.int32),
            pltpu.VMEM((_NCH, _CHUNK), jnp.int32),
            pltpu.VMEM((_CHUNK, _D), jnp.float32),
            pltpu.VMEM((_CHUNK, _D), jnp.float32),
            pltpu.SemaphoreType.DMA,
            pltpu.SemaphoreType.DMA,
            pltpu.SemaphoreType.DMA,
            pltpu.SemaphoreType.DMA,
            pltpu.VMEM_SHARED((_NPAD, _D), jnp.float32),
        ],
    )
    return f(m, src_t, dst_t, zeros)


def _mm_body(x_ref, w_ref, o_ref):
    o_ref[...] = jnp.dot(x_ref[...], w_ref[...],
                         preferred_element_type=jnp.float32)


def _dex_body(ei_ref, pad_ref, src_ref, dst_ref):
    src_ref[...] = jnp.concatenate([ei_ref[0, :], pad_ref[0, :]])
    dst_ref[...] = jnp.concatenate([ei_ref[1, :], pad_ref[1, :]])


def _dex(ei, pads):
    return pl.pallas_call(
        _dex_body,
        out_shape=(jax.ShapeDtypeStruct((_EPAD,), jnp.int32),
                   jax.ShapeDtypeStruct((_EPAD,), jnp.int32)),
    )(ei, pads)


def _gru_math(a0, a1, h, wih, whh, bih, bhh):
    agg = a0[...] + a1[...]
    hh = h[...]
    gi = lax.dot_general(agg, wih[...], (((1,), (1,)), ((), ())),
                         preferred_element_type=jnp.float32) + bih[...]
    gh = lax.dot_general(hh, whh[...], (((1,), (1,)), ((), ())),
                         preferred_element_type=jnp.float32) + bhh[...]
    r = jax.nn.sigmoid(gi[:, :_D] + gh[:, :_D])
    z = jax.nn.sigmoid(gi[:, _D:2 * _D] + gh[:, _D:2 * _D])
    n = jnp.tanh(gi[:, 2 * _D:] + r * gh[:, 2 * _D:])
    return (1.0 - z) * n + z * hh


def _gru_body(a0, a1, h, wih, whh, bih, bhh, wnext, hn_ref, mn_ref):
    hnew = _gru_math(a0, a1, h, wih, whh, bih, bhh)
    hn_ref[...] = hnew
    mn_ref[...] = jnp.dot(hnew, wnext[...], preferred_element_type=jnp.float32)


def _final_body(a0, a1, h, wih, whh, bih, bhh, linw, linb, o_ref):
    hnew = _gru_math(a0, a1, h, wih, whh, bih, bhh)
    logits = lax.dot_general(hnew, linw[...], (((1,), (1,)), ((), ())),
                             preferred_element_type=jnp.float32) + linb[...]
    mx = jnp.max(logits, axis=1, keepdims=True)
    sh = logits - mx
    o_ref[...] = sh - jnp.log(jnp.sum(jnp.exp(sh), axis=1, keepdims=True))


def _row_spec(d):
    return pl.BlockSpec((_BR, d), lambda i: (i, 0))


def _full_spec(shape):
    nd = len(shape)
    return pl.BlockSpec(shape, lambda i: (0,) * nd)


def _mm(x, w):
    return pl.pallas_call(
        _mm_body,
        grid=(_N // _BR,),
        in_specs=[_row_spec(_D), _full_spec((_D, _D))],
        out_specs=_row_spec(_D),
        out_shape=jax.ShapeDtypeStruct((_N, _D), jnp.float32),
    )(x, w)


def _gru(p0, p1, h, wih, whh, bih, bhh, wnext):
    return pl.pallas_call(
        _gru_body,
        grid=(_N // _BR,),
        in_specs=[_row_spec(_D), _row_spec(_D), _row_spec(_D),
                  _full_spec((3 * _D, _D)), _full_spec((3 * _D, _D)),
                  _full_spec((1, 3 * _D)), _full_spec((1, 3 * _D)),
                  _full_spec((_D, _D))],
        out_specs=(_row_spec(_D), _row_spec(_D)),
        out_shape=(jax.ShapeDtypeStruct((_N, _D), jnp.float32),
                   jax.ShapeDtypeStruct((_N, _D), jnp.float32)),
    )(p0, p1, h, wih, whh, bih, bhh, wnext)


def _final(p0, p1, h, wih, whh, bih, bhh, linw, linb):
    return pl.pallas_call(
        _final_body,
        grid=(_N // _BR,),
        in_specs=[_row_spec(_D), _row_spec(_D), _row_spec(_D),
                  _full_spec((3 * _D, _D)), _full_spec((3 * _D, _D)),
                  _full_spec((1, 3 * _D)), _full_spec((1, 3 * _D)),
                  _full_spec((_C, _D)), _full_spec((1, _C))],
        out_specs=_row_spec(_C),
        out_shape=jax.ShapeDtypeStruct((_N, _C), jnp.float32),
    )(p0, p1, h, wih, whh, bih, bhh, linw, linb)


def kernel(x, edge_index, W, W_ih, W_hh, b_ih, b_hh, lin_W, lin_b):
    src_p, dst_p = _dex(edge_index, _PADS)
    src_t = src_p.reshape(_NW, _NCH * _CHUNK)
    dst_t = dst_p.reshape(_NW, _NCH, _CHUNK)
    zeros = _ZEROS
    bih2 = b_ih.reshape(1, 3 * _D)
    bhh2 = b_hh.reshape(1, 3 * _D)
    linb2 = lin_b.reshape(1, _C)

    h = x
    m = _mm(x, W[0])
    for i in range(_L - 1):
        p0, p1 = _sc_scatter(m, src_t, dst_t, zeros)
        h, m = _gru(p0, p1, h, W_ih, W_hh, bih2, bhh2, W[i + 1])
    p0, p1 = _sc_scatter(m, src_t, dst_t, zeros)
    return _final(p0, p1, h, W_ih, W_hh, bih2, bhh2, lin_W, linb2)
